# baseline (device time: 19995 ns/iter reference)
import functools

import jax
import jax.numpy as jnp
from jax import lax
from jax.experimental import pallas as pl
from jax.experimental.pallas import tpu as pltpu

N_DEV = 4
NB = 2


def kernel(x, dy):
    k_per, d = x.shape
    _, f = dy.shape
    m_per = d // N_DEV
    h = m_per // 2
    fb = f // NB

    def body(x_ref, dy_ref, out_ref,
             xv_ref, dyf_ref, dyb_ref, p_ref, outv_ref,
             h2ar_send, h2al_send,
             h1r_recv, h1l_recv, r2a_recv, r2b_recv, l2a_recv, l2b_recv,
             in_sems, out_sems, send_sems, recv_sems):
        my = lax.axis_index("i")
        left = lax.rem(my + N_DEV - 1, N_DEV)
        right = lax.rem(my + 1, N_DEV)
        c2 = lax.rem(my + 2, N_DEV)
        cr = right
        cl = left

        x_cp = pltpu.make_async_copy(x_ref, xv_ref, in_sems.at[NB])
        x_cp.start()
        dy_cps = []
        for b in range(NB):
            cp = pltpu.make_async_copy(
                dy_ref.at[:, pl.ds(b * fb, fb)],
                dyf_ref.at[:, pl.ds(b * fb, fb)],
                in_sems.at[b],
            )
            cp.start()
            dy_cps.append(cp)

        barrier_sem = pltpu.get_barrier_semaphore()
        for nbr in (left, right):
            pl.semaphore_signal(
                barrier_sem, inc=1,
                device_id=(nbr,), device_id_type=pl.DeviceIdType.MESH,
            )
        pl.semaphore_wait(barrier_sem, 2)

        def cols(ref, b):
            return ref.at[:, pl.ds(b * fb, fb)]

        def pchunk(c, b):
            xs = xv_ref[:, pl.ds(c * m_per, m_per)].astype(jnp.bfloat16)
            p_ref[pl.ds(c * m_per, m_per), pl.ds(b * fb, fb)] = (
                lax.dot_general(
                    xs, dyb_ref[:, pl.ds(b * fb, fb)],
                    dimension_numbers=(((0,), (0,)), ((), ())),
                    preferred_element_type=jnp.float32,
                ).astype(jnp.bfloat16))

        def rdma(src, dst, i, dev):
            return pltpu.make_async_remote_copy(
                src_ref=src, dst_ref=dst,
                send_sem=send_sems.at[i], recv_sem=recv_sems.at[i],
                device_id=(dev,), device_id_type=pl.DeviceIdType.MESH,
            )

        x_cp.wait()
        r1 = []
        r2b = []
        for b in range(NB):
            dy_cps[b].wait()
            dyb_ref[:, pl.ds(b * fb, fb)] = (
                dyf_ref[:, pl.ds(b * fb, fb)].astype(jnp.bfloat16))

            pchunk(c2, b)
            r1r = rdma(p_ref.at[pl.ds(c2 * m_per, h), pl.ds(b * fb, fb)],
                       cols(h1r_recv, b), 6 * b + 0, right)
            r1l = rdma(p_ref.at[pl.ds(c2 * m_per + h, h), pl.ds(b * fb, fb)],
                       cols(h1l_recv, b), 6 * b + 1, left)
            r1r.start()
            r1l.start()
            r1.append((r1r, r1l))

            pchunk(cr, b)
            r2br = rdma(p_ref.at[pl.ds(cr * m_per + h, h), pl.ds(b * fb, fb)],
                        cols(r2b_recv, b), 6 * b + 2, right)
            r2br.start()
            pchunk(cl, b)
            r2bl = rdma(p_ref.at[pl.ds(cl * m_per, h), pl.ds(b * fb, fb)],
                        cols(l2b_recv, b), 6 * b + 3, left)
            r2bl.start()
            r2b.append((r2br, r2bl))

            pchunk(my, b)

        r2a = []
        for b in range(NB):
            r1r, r1l = r1[b]
            r1r.wait_recv()
            h2ar_send[:, pl.ds(b * fb, fb)] = (
                h1r_recv[:, pl.ds(b * fb, fb)]
                + p_ref[pl.ds(cr * m_per, h), pl.ds(b * fb, fb)])
            r2ar = rdma(cols(h2ar_send, b), cols(r2a_recv, b),
                        6 * b + 4, right)
            r2ar.start()

            r1l.wait_recv()
            h2al_send[:, pl.ds(b * fb, fb)] = (
                h1l_recv[:, pl.ds(b * fb, fb)]
                + p_ref[pl.ds(cl * m_per + h, h), pl.ds(b * fb, fb)])
            r2al = rdma(cols(h2al_send, b), cols(l2a_recv, b),
                        6 * b + 5, left)
            r2al.start()
            r2a.append((r2ar, r2al))

        out_cps = []
        for b in range(NB):
            r2ar, r2al = r2a[b]
            r2br, r2bl = r2b[b]
            r2ar.wait_recv()
            r2bl.wait_recv()
            outv_ref[pl.ds(0, h), pl.ds(b * fb, fb)] = (
                p_ref[pl.ds(my * m_per, h), pl.ds(b * fb, fb)]
                .astype(jnp.float32)
                + r2a_recv[:, pl.ds(b * fb, fb)].astype(jnp.float32)
                + l2b_recv[:, pl.ds(b * fb, fb)].astype(jnp.float32)
            )
            cp_t = pltpu.make_async_copy(
                outv_ref.at[pl.ds(0, h), pl.ds(b * fb, fb)],
                out_ref.at[pl.ds(0, h), pl.ds(b * fb, fb)],
                out_sems.at[2 * b],
            )
            cp_t.start()
            out_cps.append(cp_t)

            r2br.wait_recv()
            r2al.wait_recv()
            outv_ref[pl.ds(h, h), pl.ds(b * fb, fb)] = (
                p_ref[pl.ds(my * m_per + h, h), pl.ds(b * fb, fb)]
                .astype(jnp.float32)
                + r2b_recv[:, pl.ds(b * fb, fb)].astype(jnp.float32)
                + l2a_recv[:, pl.ds(b * fb, fb)].astype(jnp.float32)
            )
            cp_b = pltpu.make_async_copy(
                outv_ref.at[pl.ds(h, h), pl.ds(b * fb, fb)],
                out_ref.at[pl.ds(h, h), pl.ds(b * fb, fb)],
                out_sems.at[2 * b + 1],
            )
            cp_b.start()
            out_cps.append(cp_b)

        for cp in out_cps:
            cp.wait()

        for b in range(NB):
            for r in (*r1[b], *r2b[b], *r2a[b]):
                r.wait_send()

        @functools.partial(pl.run_scoped, sem2=pltpu.SemaphoreType.REGULAR)
        def _(sem2):
            for nbr in (left, right):
                pl.semaphore_signal(
                    sem2, inc=1,
                    device_id=(nbr,), device_id_type=pl.DeviceIdType.MESH,
                )
            pl.semaphore_wait(sem2, 2)

    hbm = pltpu.MemorySpace.HBM
    return pl.pallas_call(
        body,
        out_shape=jax.ShapeDtypeStruct((m_per, f), jnp.float32),
        in_specs=[
            pl.BlockSpec(memory_space=hbm),
            pl.BlockSpec(memory_space=hbm),
        ],
        out_specs=pl.BlockSpec(memory_space=hbm),
        scratch_shapes=[
            pltpu.VMEM((k_per, d), jnp.float32),
            pltpu.VMEM((k_per, f), jnp.float32),
            pltpu.VMEM((k_per, f), jnp.bfloat16),
            pltpu.VMEM((d, f), jnp.bfloat16),
            pltpu.VMEM((m_per, f), jnp.float32),
            pltpu.VMEM((h, f), jnp.bfloat16),
            pltpu.VMEM((h, f), jnp.bfloat16),
            pltpu.VMEM((h, f), jnp.bfloat16),
            pltpu.VMEM((h, f), jnp.bfloat16),
            pltpu.VMEM((h, f), jnp.bfloat16),
            pltpu.VMEM((h, f), jnp.bfloat16),
            pltpu.VMEM((h, f), jnp.bfloat16),
            pltpu.VMEM((h, f), jnp.bfloat16),
            pltpu.SemaphoreType.DMA((NB + 1,)),
            pltpu.SemaphoreType.DMA((2 * NB,)),
            pltpu.SemaphoreType.DMA((6 * NB,)),
            pltpu.SemaphoreType.DMA((6 * NB,)),
        ],
        compiler_params=pltpu.CompilerParams(collective_id=0),
    )(x, dy)


# device time: 17880 ns/iter; 1.1183x vs baseline; 1.1183x over previous
import jax
import jax.numpy as jnp
from jax import lax
from jax.experimental import pallas as pl
from jax.experimental.pallas import tpu as pltpu

N_DEV = 4


def kernel(x, dy):
    k_per, d = x.shape
    _, f = dy.shape
    m_per = d // N_DEV
    h = m_per // 2

    xb = x.astype(jnp.bfloat16)
    dyb = dy.astype(jnp.bfloat16)

    def body(x_ref, dy_ref, out_ref, p_ref,
             h2ar_send, h2al_send,
             h1r_recv, h1l_recv, r2a_recv, r2b_recv, l2a_recv, l2b_recv,
             send_sems, recv_sems):
        my = lax.axis_index("i")
        left = lax.rem(my + N_DEV - 1, N_DEV)
        right = lax.rem(my + 1, N_DEV)
        c2 = lax.rem(my + 2, N_DEV)
        cr = right
        cl = left

        barrier_sem = pltpu.get_barrier_semaphore()
        for nbr in (left, right):
            pl.semaphore_signal(
                barrier_sem, inc=1,
                device_id=(nbr,), device_id_type=pl.DeviceIdType.MESH,
            )
        pl.semaphore_wait(barrier_sem, 2)

        def pchunk(c):
            p_ref[pl.ds(c * m_per, m_per), :] = lax.dot_general(
                x_ref[:, pl.ds(c * m_per, m_per)], dy_ref[:, :],
                dimension_numbers=(((0,), (0,)), ((), ())),
                preferred_element_type=jnp.float32,
            ).astype(jnp.bfloat16)

        def rdma(src, dst, i, dev):
            return pltpu.make_async_remote_copy(
                src_ref=src, dst_ref=dst,
                send_sem=send_sems.at[i], recv_sem=recv_sems.at[i],
                device_id=(dev,), device_id_type=pl.DeviceIdType.MESH,
            )

        pchunk(c2)
        r1r = rdma(p_ref.at[pl.ds(c2 * m_per, h)], h1r_recv, 0, right)
        r1l = rdma(p_ref.at[pl.ds(c2 * m_per + h, h)], h1l_recv, 1, left)
        r1r.start()
        r1l.start()

        pchunk(cr)
        r2br = rdma(p_ref.at[pl.ds(cr * m_per + h, h)], r2b_recv, 2, right)
        r2br.start()
        pchunk(cl)
        r2bl = rdma(p_ref.at[pl.ds(cl * m_per, h)], l2b_recv, 3, left)
        r2bl.start()

        pchunk(my)

        r1r.wait_recv()
        h2ar_send[:, :] = h1r_recv[:, :] + p_ref[pl.ds(cr * m_per, h), :]
        r2ar = rdma(h2ar_send, r2a_recv, 4, right)
        r2ar.start()

        r1l.wait_recv()
        h2al_send[:, :] = h1l_recv[:, :] + p_ref[pl.ds(cl * m_per + h, h), :]
        r2al = rdma(h2al_send, l2a_recv, 5, left)
        r2al.start()

        r2ar.wait_recv()
        r2bl.wait_recv()
        out_ref[pl.ds(0, h), :] = (
            p_ref[pl.ds(my * m_per, h), :].astype(jnp.float32)
            + r2a_recv[:, :].astype(jnp.float32)
            + l2b_recv[:, :].astype(jnp.float32)
        )
        r2br.wait_recv()
        r2al.wait_recv()
        out_ref[pl.ds(h, h), :] = (
            p_ref[pl.ds(my * m_per + h, h), :].astype(jnp.float32)
            + r2b_recv[:, :].astype(jnp.float32)
            + l2a_recv[:, :].astype(jnp.float32)
        )

        for r in (r1r, r1l, r2br, r2bl, r2ar, r2al):
            r.wait_send()

    return pl.pallas_call(
        body,
        out_shape=jax.ShapeDtypeStruct((m_per, f), jnp.float32),
        in_specs=[
            pl.BlockSpec(memory_space=pltpu.VMEM),
            pl.BlockSpec(memory_space=pltpu.VMEM),
        ],
        out_specs=pl.BlockSpec(memory_space=pltpu.VMEM),
        scratch_shapes=[
            pltpu.VMEM((d, f), jnp.bfloat16),
            pltpu.VMEM((h, f), jnp.bfloat16),
            pltpu.VMEM((h, f), jnp.bfloat16),
            pltpu.VMEM((h, f), jnp.bfloat16),
            pltpu.VMEM((h, f), jnp.bfloat16),
            pltpu.VMEM((h, f), jnp.bfloat16),
            pltpu.VMEM((h, f), jnp.bfloat16),
            pltpu.VMEM((h, f), jnp.bfloat16),
            pltpu.VMEM((h, f), jnp.bfloat16),
            pltpu.SemaphoreType.DMA((6,)),
            pltpu.SemaphoreType.DMA((6,)),
        ],
        compiler_params=pltpu.CompilerParams(collective_id=0),
    )(xb, dyb)


# device time: 17681 ns/iter; 1.1309x vs baseline; 1.0113x over previous
import jax
import jax.numpy as jnp
from jax import lax
from jax.experimental import pallas as pl
from jax.experimental.pallas import tpu as pltpu

N_DEV = 4


def kernel(x, dy):
    k_per, d = x.shape
    _, f = dy.shape
    m_per = d // N_DEV
    h = m_per // 2

    dyb = dy.astype(jnp.bfloat16)

    def body(x_ref, dy_ref, out_ref, p_ref,
             h2ar_send, h2al_send,
             h1r_recv, h1l_recv, r2a_recv, r2b_recv, l2a_recv, l2b_recv,
             tmp_top, tmp_bot,
             send_sems, recv_sems):
        my = lax.axis_index("i")
        left = lax.rem(my + N_DEV - 1, N_DEV)
        right = lax.rem(my + 1, N_DEV)
        c2 = lax.rem(my + 2, N_DEV)
        cr = right
        cl = left

        barrier_sem = pltpu.get_barrier_semaphore()
        for nbr in (left, right):
            pl.semaphore_signal(
                barrier_sem, inc=1,
                device_id=(nbr,), device_id_type=pl.DeviceIdType.MESH,
            )
        pl.semaphore_wait(barrier_sem, 2)

        def pchunk(c):
            p_ref[pl.ds(c * m_per, m_per), :] = lax.dot_general(
                x_ref[:, pl.ds(c * m_per, m_per)].astype(jnp.bfloat16),
                dy_ref[:, :],
                dimension_numbers=(((0,), (0,)), ((), ())),
                preferred_element_type=jnp.float32,
            ).astype(jnp.bfloat16)

        def rdma(src, dst, i, dev):
            return pltpu.make_async_remote_copy(
                src_ref=src, dst_ref=dst,
                send_sem=send_sems.at[i], recv_sem=recv_sems.at[i],
                device_id=(dev,), device_id_type=pl.DeviceIdType.MESH,
            )

        pchunk(c2)
        r1r = rdma(p_ref.at[pl.ds(c2 * m_per, h)], h1r_recv, 0, right)
        r1l = rdma(p_ref.at[pl.ds(c2 * m_per + h, h)], h1l_recv, 1, left)
        r1r.start()
        r1l.start()

        pchunk(cr)
        r2br = rdma(p_ref.at[pl.ds(cr * m_per + h, h)], r2b_recv, 2, right)
        r2br.start()
        pchunk(cl)
        r2bl = rdma(p_ref.at[pl.ds(cl * m_per, h)], l2b_recv, 3, left)
        r2bl.start()

        pchunk(my)

        r1r.wait_recv()
        h2ar_send[:, :] = h1r_recv[:, :] + p_ref[pl.ds(cr * m_per, h), :]
        r2ar = rdma(h2ar_send, r2a_recv, 4, right)
        r2ar.start()

        r1l.wait_recv()
        h2al_send[:, :] = h1l_recv[:, :] + p_ref[pl.ds(cl * m_per + h, h), :]
        r2al = rdma(h2al_send, l2a_recv, 5, left)
        r2al.start()

        r2br.wait_recv()
        r2bl.wait_recv()
        tmp_top[:, :] = (
            p_ref[pl.ds(my * m_per, h), :].astype(jnp.float32)
            + l2b_recv[:, :].astype(jnp.float32))
        tmp_bot[:, :] = (
            p_ref[pl.ds(my * m_per + h, h), :].astype(jnp.float32)
            + r2b_recv[:, :].astype(jnp.float32))
        r2ar.wait_recv()
        out_ref[pl.ds(0, h), :] = (
            tmp_top[:, :] + r2a_recv[:, :].astype(jnp.float32))
        r2al.wait_recv()
        out_ref[pl.ds(h, h), :] = (
            tmp_bot[:, :] + l2a_recv[:, :].astype(jnp.float32))

        for r in (r1r, r1l, r2br, r2bl, r2ar, r2al):
            r.wait_send()

    return pl.pallas_call(
        body,
        out_shape=jax.ShapeDtypeStruct((m_per, f), jnp.float32),
        in_specs=[
            pl.BlockSpec(memory_space=pltpu.VMEM),
            pl.BlockSpec(memory_space=pltpu.VMEM),
        ],
        out_specs=pl.BlockSpec(memory_space=pltpu.VMEM),
        scratch_shapes=[
            pltpu.VMEM((d, f), jnp.bfloat16),
            pltpu.VMEM((h, f), jnp.bfloat16),
            pltpu.VMEM((h, f), jnp.bfloat16),
            pltpu.VMEM((h, f), jnp.bfloat16),
            pltpu.VMEM((h, f), jnp.bfloat16),
            pltpu.VMEM((h, f), jnp.bfloat16),
            pltpu.VMEM((h, f), jnp.bfloat16),
            pltpu.VMEM((h, f), jnp.bfloat16),
            pltpu.VMEM((h, f), jnp.bfloat16),
            pltpu.VMEM((h, f), jnp.float32),
            pltpu.VMEM((h, f), jnp.float32),
            pltpu.SemaphoreType.DMA((6,)),
            pltpu.SemaphoreType.DMA((6,)),
        ],
        compiler_params=pltpu.CompilerParams(collective_id=0),
    )(x, dyb)
